# baseline (speedup 1.0000x reference)
import functools

import jax
import jax.numpy as jnp
from jax import lax
from jax.experimental import pallas as pl
from jax.experimental.pallas import tpu as pltpu
from jax.experimental.pallas import tpu_sc as plsc

K = 16
QB = 128
NCH = 128
NW = 32
B, N, DIM, Q = 4, 16384, 64, 2048
ROWS = B * Q
RPW = ROWS // NW
CHL = N // NCH


def _score_block(ref_ref, q_ref, s_ref, cm_ref):
    reft = ref_ref[0]
    q = q_ref[0]
    pcsq = jnp.sum(reft * reft, axis=0, keepdims=True)
    qsq = jnp.sum(q * q, axis=1, keepdims=True)
    qp = lax.dot_general(q.astype(jnp.bfloat16), reft.astype(jnp.bfloat16),
                         (((1,), (0,)), ((), ())),
                         preferred_element_type=jnp.float32)
    s = (qsq - 2.0 * qp) + pcsq
    s_ref[...] = s
    m = s
    while m.shape[1] > NCH:
        w = m.shape[1] // 2
        m = jnp.minimum(m[:, :w], m[:, w:])
    cm_ref[...] = m


@jax.jit
def _scores(reft, query):
    qpb = Q // QB
    grid = (B, qpb)
    return pl.pallas_call(
        _score_block,
        grid=grid,
        in_specs=[
            pl.BlockSpec((1, DIM, N), lambda bi, qi: (bi, 0, 0)),
            pl.BlockSpec((1, QB, DIM), lambda bi, qi: (bi, qi, 0)),
        ],
        out_specs=[
            pl.BlockSpec((QB, N), lambda bi, qi: (bi * qpb + qi, 0)),
            pl.BlockSpec((QB, NCH), lambda bi, qi: (bi * qpb + qi, 0)),
        ],
        out_shape=[
            jax.ShapeDtypeStruct((ROWS, N), jnp.float32),
            jax.ShapeDtypeStruct((ROWS, NCH), jnp.float32),
        ],
    )(reft, query)


def _merge16(rv, ri, cv, ci):
    cs, cis = plsc.sort_key_val(cv, ci)
    cr = jnp.flip(cs, 0)
    cir = jnp.flip(cis, 0)
    take = cr < rv
    nv = jnp.where(take, cr, rv)
    ni = jnp.where(take, cir, ri)
    sv, si = plsc.sort_key_val(nv, ni)
    return sv, si


def _nsqrt(x):
    xb = lax.bitcast_convert_type(x, jnp.int32)
    y = lax.bitcast_convert_type(
        (xb >> 1) + jnp.int32(0x1FBD1DF5), jnp.float32)
    y = 0.5 * (y + x / y)
    y = 0.5 * (y + x / y)
    y = 0.5 * (y + x / y)
    return y


def _sc_body(s_hbm, cm_hbm, d_hbm, i_hbm,
             rowb0, rowb1, cmb0, cmb1, selb, scmb, odb, oib,
             rs0, rs1, cs0, cs1, osem):
    wid = lax.axis_index("s") * 2 + lax.axis_index("c")
    base = wid * RPW
    rowbs = (rowb0, rowb1)
    cmbs = (cmb0, cmb1)
    rsems = (rs0, rs1)
    csems = (cs0, cs1)
    lane = lax.broadcasted_iota(jnp.int32, (16,), 0)
    inf16 = jnp.full((16,), jnp.inf, jnp.float32)
    zero16 = jnp.zeros((16,), jnp.int32)

    def row_cp(r, bf):
        return pltpu.make_async_copy(s_hbm.at[base + r], rowbs[bf],
                                     rsems[bf])

    def cm_cp(r, bf):
        return pltpu.make_async_copy(cm_hbm.at[base + r], cmbs[bf],
                                     csems[bf])

    row_cp(0, 0).start()
    cm_cp(0, 0).start()
    row_cp(1, 1).start()
    cm_cp(1, 1).start()

    lane128 = lane * NCH

    def _mlow(a, b):
        m = jnp.minimum(a, jnp.flip(b, 0))
        s, _ = plsc.sort_key_val(m, m)
        return s

    def process(r, bf):
        cmr = cmbs[bf]
        rowv = rowbs[bf]
        lv = [plsc.sort_key_val(cmr[pl.ds(g * 16, 16)],
                                zero16)[0] for g in range(8)]
        l1 = [_mlow(lv[0], lv[1]), _mlow(lv[2], lv[3]),
              _mlow(lv[4], lv[5]), _mlow(lv[6], lv[7])]
        l2 = [_mlow(l1[0], l1[1]), _mlow(l1[2], l1[3])]
        t16 = _mlow(l2[0], l2[1])
        tb = jnp.broadcast_to(t16[15], (16,))
        nsel = jnp.int32(0)
        for g in range(NCH // 16):
            cv = cmr[pl.ds(g * 16, 16)]
            msk = cv <= tb
            plsc.store_compressed(selb.at[pl.ds(nsel, 16)], lane + g * 16,
                                  mask=msk)
            plsc.store_compressed(scmb.at[pl.ds(nsel, 16)], cv, mask=msk)
            nsel = nsel + plsc.all_reduce_population_count(msk)[0]

        def chunk_body(i, carry):
            rv, ri = carry
            cid = selb[pl.ds(i, 16)][0]
            cmv = scmb[pl.ds(i, 16)][0]

            def scan_chunk(args):
                rv, ri = args
                r15b = jnp.broadcast_to(rv[15], (16,))
                for g in range(CHL // 16):
                    idx = cid + lane128 + g * (16 * NCH)
                    cv = plsc.load_gather(rowv, [idx])
                    hit = jnp.any(cv < r15b)

                    def do_merge(a):
                        return _merge16(*a)

                    def skip(a):
                        return a[0], a[1]

                    rv, ri = lax.cond(hit, do_merge, skip, (rv, ri, cv, idx))
                return rv, ri

            def skip_chunk(args):
                return args

            return lax.cond(cmv < rv[15], scan_chunk, skip_chunk, (rv, ri))

        rv, ri = lax.fori_loop(0, nsel, chunk_body, (inf16, zero16))

        d = _nsqrt(jnp.maximum(rv, 1e-12))
        odb[pl.ds(r * 16, 16)] = d
        oib[pl.ds(r * 16, 16)] = ri

    def outer(rr, _):
        for bf in range(2):
            r = rr * 2 + bf
            row_cp(r, bf).wait()
            cm_cp(r, bf).wait()
            process(r, bf)
            nxt = r + 2

            @pl.when(nxt < RPW)
            def _():
                row_cp(nxt, bf).start()
                cm_cp(nxt, bf).start()
        return 0

    lax.fori_loop(0, RPW // 2, outer, 0)
    pltpu.make_async_copy(odb, d_hbm.at[pl.ds(base * 16, RPW * 16)],
                          osem).start()
    pltpu.make_async_copy(odb, d_hbm.at[pl.ds(base * 16, RPW * 16)],
                          osem).wait()
    pltpu.sync_copy(oib, i_hbm.at[pl.ds(base * 16, RPW * 16)])


@jax.jit
def _sc_topk(s_mat, cm_mat):
    mesh = plsc.VectorSubcoreMesh(core_axis_name="c", subcore_axis_name="s")
    f = pl.kernel(
        _sc_body,
        out_type=[
            jax.ShapeDtypeStruct((ROWS * K,), jnp.float32),
            jax.ShapeDtypeStruct((ROWS * K,), jnp.int32),
        ],
        mesh=mesh,
        compiler_params=pltpu.CompilerParams(needs_layout_passes=False),
        scratch_types=[
            pltpu.VMEM((N,), jnp.float32),
            pltpu.VMEM((N,), jnp.float32),
            pltpu.VMEM((NCH,), jnp.float32),
            pltpu.VMEM((NCH,), jnp.float32),
            pltpu.VMEM((NCH + 32,), jnp.int32),
            pltpu.VMEM((NCH + 32,), jnp.float32),
            pltpu.VMEM((RPW * K,), jnp.float32),
            pltpu.VMEM((RPW * K,), jnp.int32),
            pltpu.SemaphoreType.DMA,
            pltpu.SemaphoreType.DMA,
            pltpu.SemaphoreType.DMA,
            pltpu.SemaphoreType.DMA,
            pltpu.SemaphoreType.DMA,
        ],
    )
    return f(s_mat, cm_mat)


def kernel(ref, query):
    s_mat, cm_mat = _scores(jnp.swapaxes(ref, 1, 2), query)
    d_flat, i_flat = _sc_topk(s_mat, cm_mat)
    dist = d_flat.reshape(B, Q, K)
    idx = i_flat.reshape(B, Q, K)
    return dist, idx.astype(jnp.int64)


# speedup vs baseline: 1.1354x; 1.1354x over previous
import functools

import jax
import jax.numpy as jnp
from jax import lax
from jax.experimental import pallas as pl
from jax.experimental.pallas import tpu as pltpu
from jax.experimental.pallas import tpu_sc as plsc

K = 16
QB = 128
NCH = 128
NW = 32
B, N, DIM, Q = 4, 16384, 64, 2048
ROWS = B * Q
RPW = ROWS // NW
CHL = N // NCH


def _score_block(ref_ref, q_ref, s_ref, cm_ref):
    reft = ref_ref[0]
    q = q_ref[0]
    pcsq = jnp.sum(reft * reft, axis=0, keepdims=True)
    qsq = jnp.sum(q * q, axis=1, keepdims=True)
    qp = lax.dot_general(q.astype(jnp.bfloat16), reft.astype(jnp.bfloat16),
                         (((1,), (0,)), ((), ())),
                         preferred_element_type=jnp.float32)
    s = (qsq - 2.0 * qp) + pcsq
    s_ref[...] = s
    m = s
    while m.shape[1] > NCH:
        w = m.shape[1] // 2
        m = jnp.minimum(m[:, :w], m[:, w:])
    cm_ref[...] = m


@jax.jit
def _scores(reft, query):
    qpb = Q // QB
    grid = (B, qpb)
    return pl.pallas_call(
        _score_block,
        grid=grid,
        in_specs=[
            pl.BlockSpec((1, DIM, N), lambda bi, qi: (bi, 0, 0)),
            pl.BlockSpec((1, QB, DIM), lambda bi, qi: (bi, qi, 0)),
        ],
        out_specs=[
            pl.BlockSpec((QB, N), lambda bi, qi: (bi * qpb + qi, 0)),
            pl.BlockSpec((QB, NCH), lambda bi, qi: (bi * qpb + qi, 0)),
        ],
        out_shape=[
            jax.ShapeDtypeStruct((ROWS, N), jnp.float32),
            jax.ShapeDtypeStruct((ROWS, NCH), jnp.float32),
        ],
    )(reft, query)


def _merge16(rv, ri, cv, ci):
    cs, cis = plsc.sort_key_val(cv, ci)
    cr = jnp.flip(cs, 0)
    cir = jnp.flip(cis, 0)
    take = cr < rv
    nv = jnp.where(take, cr, rv)
    ni = jnp.where(take, cir, ri)
    sv, si = plsc.sort_key_val(nv, ni)
    return sv, si


def _nsqrt(x):
    xb = lax.bitcast_convert_type(x, jnp.int32)
    y = lax.bitcast_convert_type(
        (xb >> 1) + jnp.int32(0x1FBD1DF5), jnp.float32)
    y = 0.5 * (y + x / y)
    y = 0.5 * (y + x / y)
    y = 0.5 * (y + x / y)
    return y


def _sc_body(s_hbm, cm_hbm, d_hbm, i_hbm,
             rowb0, rowb1, cmb0, cmb1, selb, scmb, odb, oib,
             rs0, rs1, cs0, cs1, osem):
    wid = lax.axis_index("s") * 2 + lax.axis_index("c")
    base = wid * RPW
    rowbs = (rowb0, rowb1)
    cmbs = (cmb0, cmb1)
    rsems = (rs0, rs1)
    csems = (cs0, cs1)
    lane = lax.broadcasted_iota(jnp.int32, (16,), 0)
    inf16 = jnp.full((16,), jnp.inf, jnp.float32)
    zero16 = jnp.zeros((16,), jnp.int32)

    def row_cp(r, bf):
        return pltpu.make_async_copy(s_hbm.at[base + r], rowbs[bf],
                                     rsems[bf])

    def cm_cp(r, bf):
        return pltpu.make_async_copy(cm_hbm.at[base + r], cmbs[bf],
                                     csems[bf])

    row_cp(0, 0).start()
    cm_cp(0, 0).start()
    row_cp(1, 1).start()
    cm_cp(1, 1).start()

    lane128 = lane * NCH

    def _mlow(a, b):
        m = jnp.minimum(a, jnp.flip(b, 0))
        s, _ = plsc.sort_key_val(m, m)
        return s

    def process(r, bf):
        cmr = cmbs[bf]
        rowv = rowbs[bf]
        lv = [plsc.sort_key_val(cmr[pl.ds(g * 16, 16)],
                                zero16)[0] for g in range(8)]
        l1 = [_mlow(lv[0], lv[1]), _mlow(lv[2], lv[3]),
              _mlow(lv[4], lv[5]), _mlow(lv[6], lv[7])]
        l2 = [_mlow(l1[0], l1[1]), _mlow(l1[2], l1[3])]
        t16 = _mlow(l2[0], l2[1])
        tb = jnp.broadcast_to(t16[15], (16,))
        nsel = jnp.int32(0)
        for g in range(NCH // 16):
            cv = cmr[pl.ds(g * 16, 16)]
            msk = cv <= tb
            plsc.store_compressed(selb.at[pl.ds(nsel, 16)], lane + g * 16,
                                  mask=msk)
            plsc.store_compressed(scmb.at[pl.ds(nsel, 16)], cv, mask=msk)
            nsel = nsel + plsc.all_reduce_population_count(msk)[0]

        def _sel2(c, a, b):
            return jnp.where(c, a[0], b[0]), jnp.where(c, a[1], b[1])

        def _pair(a, b):
            c = a[0] <= b[0]
            return _sel2(c, a, b), _sel2(c, b, a)

        def _comb(x, y):
            m1, m2 = x
            n1, n2 = y
            c = m1[0] <= n1[0]
            w1 = _sel2(c, m1, n1)
            lose = _sel2(c, n1, m1)
            d = m2[0] <= n2[0]
            u = _sel2(d, m2, n2)
            e = lose[0] <= u[0]
            return w1, _sel2(e, lose, u)

        def do_merge(a):
            return _merge16(*a)

        def skip(a):
            return a[0], a[1]

        def chunk_body(i, carry):
            rv, ri = carry
            cid = selb[pl.ds(i, 16)][0]
            cmv = scmb[pl.ds(i, 16)][0]

            def scan_chunk(args):
                rv, ri = args
                vs = []
                for g in range(CHL // 16):
                    idx = cid + lane128 + g * (16 * NCH)
                    vs.append((plsc.load_gather(rowv, [idx]), idx))
                p = [_pair(vs[0], vs[1]), _pair(vs[2], vs[3]),
                     _pair(vs[4], vs[5]), _pair(vs[6], vs[7])]
                m1, m2 = _comb(_comb(p[0], p[1]), _comb(p[2], p[3]))
                r15b = jnp.broadcast_to(rv[15], (16,))
                rv, ri = lax.cond(jnp.any(m1[0] < r15b), do_merge, skip,
                                  (rv, ri, m1[0], m1[1]))
                r15c = jnp.broadcast_to(rv[15], (16,))
                rv, ri = lax.cond(jnp.any(m2[0] < r15c), do_merge, skip,
                                  (rv, ri, m2[0], m2[1]))
                return rv, ri

            def skip_chunk(args):
                return args

            return lax.cond(cmv < rv[15], scan_chunk, skip_chunk, (rv, ri))

        rv, ri = lax.fori_loop(0, nsel, chunk_body, (inf16, zero16))

        d = _nsqrt(jnp.maximum(rv, 1e-12))
        odb[pl.ds(r * 16, 16)] = d
        oib[pl.ds(r * 16, 16)] = ri

    def outer(rr, _):
        for bf in range(2):
            r = rr * 2 + bf
            row_cp(r, bf).wait()
            cm_cp(r, bf).wait()
            process(r, bf)
            nxt = r + 2

            @pl.when(nxt < RPW)
            def _():
                row_cp(nxt, bf).start()
                cm_cp(nxt, bf).start()
        return 0

    lax.fori_loop(0, RPW // 2, outer, 0)
    pltpu.make_async_copy(odb, d_hbm.at[pl.ds(base * 16, RPW * 16)],
                          osem).start()
    pltpu.make_async_copy(odb, d_hbm.at[pl.ds(base * 16, RPW * 16)],
                          osem).wait()
    pltpu.sync_copy(oib, i_hbm.at[pl.ds(base * 16, RPW * 16)])


@jax.jit
def _sc_topk(s_mat, cm_mat):
    mesh = plsc.VectorSubcoreMesh(core_axis_name="c", subcore_axis_name="s")
    f = pl.kernel(
        _sc_body,
        out_type=[
            jax.ShapeDtypeStruct((ROWS * K,), jnp.float32),
            jax.ShapeDtypeStruct((ROWS * K,), jnp.int32),
        ],
        mesh=mesh,
        compiler_params=pltpu.CompilerParams(needs_layout_passes=False),
        scratch_types=[
            pltpu.VMEM((N,), jnp.float32),
            pltpu.VMEM((N,), jnp.float32),
            pltpu.VMEM((NCH,), jnp.float32),
            pltpu.VMEM((NCH,), jnp.float32),
            pltpu.VMEM((NCH + 32,), jnp.int32),
            pltpu.VMEM((NCH + 32,), jnp.float32),
            pltpu.VMEM((RPW * K,), jnp.float32),
            pltpu.VMEM((RPW * K,), jnp.int32),
            pltpu.SemaphoreType.DMA,
            pltpu.SemaphoreType.DMA,
            pltpu.SemaphoreType.DMA,
            pltpu.SemaphoreType.DMA,
            pltpu.SemaphoreType.DMA,
        ],
    )
    return f(s_mat, cm_mat)


def kernel(ref, query):
    s_mat, cm_mat = _scores(jnp.swapaxes(ref, 1, 2), query)
    d_flat, i_flat = _sc_topk(s_mat, cm_mat)
    dist = d_flat.reshape(B, Q, K)
    idx = i_flat.reshape(B, Q, K)
    return dist, idx.astype(jnp.int64)


# speedup vs baseline: 1.3818x; 1.2170x over previous
import functools

import jax
import jax.numpy as jnp
from jax import lax
from jax.experimental import pallas as pl
from jax.experimental.pallas import tpu as pltpu
from jax.experimental.pallas import tpu_sc as plsc

K = 16
QB = 128
NCH = 128
NW = 32
B, N, DIM, Q = 4, 16384, 64, 2048
ROWS = B * Q
RPW = ROWS // NW
CHL = N // NCH


def _score_block(ref_ref, q_ref, s_ref, cm_ref):
    reft = ref_ref[0]
    q = q_ref[0]
    pcsq = jnp.sum(reft * reft, axis=0, keepdims=True)
    qsq = jnp.sum(q * q, axis=1, keepdims=True)
    qp = lax.dot_general(q.astype(jnp.bfloat16), reft.astype(jnp.bfloat16),
                         (((1,), (0,)), ((), ())),
                         preferred_element_type=jnp.float32)
    s = (qsq - 2.0 * qp) + pcsq
    s_ref[...] = s
    m = s
    while m.shape[1] > NCH:
        w = m.shape[1] // 2
        m = jnp.minimum(m[:, :w], m[:, w:])
    cm_ref[...] = m


@jax.jit
def _scores(reft, query):
    qpb = Q // QB
    grid = (B, qpb)
    return pl.pallas_call(
        _score_block,
        grid=grid,
        in_specs=[
            pl.BlockSpec((1, DIM, N), lambda bi, qi: (bi, 0, 0)),
            pl.BlockSpec((1, QB, DIM), lambda bi, qi: (bi, qi, 0)),
        ],
        out_specs=[
            pl.BlockSpec((QB, N), lambda bi, qi: (bi * qpb + qi, 0)),
            pl.BlockSpec((QB, NCH), lambda bi, qi: (bi * qpb + qi, 0)),
        ],
        out_shape=[
            jax.ShapeDtypeStruct((ROWS, N), jnp.float32),
            jax.ShapeDtypeStruct((ROWS, NCH), jnp.float32),
        ],
    )(reft, query)


def _merge16(rv, ri, cv, ci):
    cs, cis = plsc.sort_key_val(cv, ci)
    cr = jnp.flip(cs, 0)
    cir = jnp.flip(cis, 0)
    take = cr < rv
    nv = jnp.where(take, cr, rv)
    ni = jnp.where(take, cir, ri)
    sv, si = plsc.sort_key_val(nv, ni)
    return sv, si


def _nsqrt(x):
    xb = lax.bitcast_convert_type(x, jnp.int32)
    y = lax.bitcast_convert_type(
        (xb >> 1) + jnp.int32(0x1FBD1DF5), jnp.float32)
    y = 0.5 * (y + x / y)
    y = 0.5 * (y + x / y)
    y = 0.5 * (y + x / y)
    return y


def _sc_body(s_hbm, cm_hbm, d_hbm, i_hbm,
             rowb0, rowb1, cmb0, cmb1, odb, oib,
             rs0, rs1, cs0, cs1, osem):
    wid = lax.axis_index("s") * 2 + lax.axis_index("c")
    base = wid * RPW
    rowbs = (rowb0, rowb1)
    cmbs = (cmb0, cmb1)
    rsems = (rs0, rs1)
    csems = (cs0, cs1)
    lane = lax.broadcasted_iota(jnp.int32, (16,), 0)
    inf16 = jnp.full((16,), jnp.inf, jnp.float32)
    zero16 = jnp.zeros((16,), jnp.int32)

    def row_cp(r, bf):
        return pltpu.make_async_copy(s_hbm.at[base + r], rowbs[bf],
                                     rsems[bf])

    def cm_cp(r, bf):
        return pltpu.make_async_copy(cm_hbm.at[base + r], cmbs[bf],
                                     csems[bf])

    row_cp(0, 0).start()
    cm_cp(0, 0).start()
    row_cp(1, 1).start()
    cm_cp(1, 1).start()

    lane128 = lane * NCH

    def _mlow(a, b):
        m = jnp.minimum(a, jnp.flip(b, 0))
        s, _ = plsc.sort_key_val(m, m)
        return s

    def _mlowi(av, ai, bv, bi):
        rbv = jnp.flip(bv, 0)
        rbi = jnp.flip(bi, 0)
        c = rbv < av
        sv, si = plsc.sort_key_val(jnp.where(c, rbv, av),
                                   jnp.where(c, rbi, ai))
        return sv, si

    def process(r, bf):
        cmr = cmbs[bf]
        rowv = rowbs[bf]
        lv = [plsc.sort_key_val(cmr[pl.ds(g * 16, 16)], lane + g * 16)
              for g in range(8)]
        l1 = [_mlowi(*lv[0], *lv[1]), _mlowi(*lv[2], *lv[3]),
              _mlowi(*lv[4], *lv[5]), _mlowi(*lv[6], *lv[7])]
        l2 = [_mlowi(*l1[0], *l1[1]), _mlowi(*l1[2], *l1[3])]
        t16v, t16i = _mlowi(*l2[0], *l2[1])

        def _sel2(c, a, b):
            return jnp.where(c, a[0], b[0]), jnp.where(c, a[1], b[1])

        def _pair(a, b):
            c = a[0] <= b[0]
            return _sel2(c, a, b), _sel2(c, b, a)

        def _comb(x, y):
            m1, m2 = x
            n1, n2 = y
            c = m1[0] <= n1[0]
            w1 = _sel2(c, m1, n1)
            lose = _sel2(c, n1, m1)
            d = m2[0] <= n2[0]
            u = _sel2(d, m2, n2)
            e = lose[0] <= u[0]
            return w1, _sel2(e, lose, u)

        rv, ri = inf16, zero16
        s2v, s2i = inf16, zero16
        for j in range(16):
            cmj = t16v[j]
            cidj = t16i[j]

            def scan_chunk(args):
                rv, ri, s2v, s2i = args
                vs = []
                for g in range(CHL // 16):
                    idx = cidj + lane128 + g * (16 * NCH)
                    vs.append((plsc.load_gather(rowv, [idx]), idx))
                p = [_pair(vs[0], vs[1]), _pair(vs[2], vs[3]),
                     _pair(vs[4], vs[5]), _pair(vs[6], vs[7])]
                m1, m2 = _comb(_comb(p[0], p[1]), _comb(p[2], p[3]))
                rv2, ri2 = _merge16(rv, ri, m1[0], m1[1])
                c2 = m2[0] < s2v
                return (rv2, ri2, jnp.where(c2, m2[0], s2v),
                        jnp.where(c2, m2[1], s2i))

            def skip_chunk(args):
                return args

            rv, ri, s2v, s2i = lax.cond(cmj < rv[15], scan_chunk,
                                        skip_chunk, (rv, ri, s2v, s2i))
        rv, ri = _merge16(rv, ri, s2v, s2i)

        d = _nsqrt(jnp.maximum(rv, 1e-12))
        odb[pl.ds(r * 16, 16)] = d
        oib[pl.ds(r * 16, 16)] = ri

    def outer(rr, _):
        for bf in range(2):
            r = rr * 2 + bf
            row_cp(r, bf).wait()
            cm_cp(r, bf).wait()
            process(r, bf)
            nxt = r + 2

            @pl.when(nxt < RPW)
            def _():
                row_cp(nxt, bf).start()
                cm_cp(nxt, bf).start()
        return 0

    lax.fori_loop(0, RPW // 2, outer, 0)
    pltpu.make_async_copy(odb, d_hbm.at[pl.ds(base * 16, RPW * 16)],
                          osem).start()
    pltpu.make_async_copy(odb, d_hbm.at[pl.ds(base * 16, RPW * 16)],
                          osem).wait()
    pltpu.sync_copy(oib, i_hbm.at[pl.ds(base * 16, RPW * 16)])


@jax.jit
def _sc_topk(s_mat, cm_mat):
    mesh = plsc.VectorSubcoreMesh(core_axis_name="c", subcore_axis_name="s")
    f = pl.kernel(
        _sc_body,
        out_type=[
            jax.ShapeDtypeStruct((ROWS * K,), jnp.float32),
            jax.ShapeDtypeStruct((ROWS * K,), jnp.int32),
        ],
        mesh=mesh,
        compiler_params=pltpu.CompilerParams(needs_layout_passes=False),
        scratch_types=[
            pltpu.VMEM((N,), jnp.float32),
            pltpu.VMEM((N,), jnp.float32),
            pltpu.VMEM((NCH,), jnp.float32),
            pltpu.VMEM((NCH,), jnp.float32),
            pltpu.VMEM((RPW * K,), jnp.float32),
            pltpu.VMEM((RPW * K,), jnp.int32),
            pltpu.SemaphoreType.DMA,
            pltpu.SemaphoreType.DMA,
            pltpu.SemaphoreType.DMA,
            pltpu.SemaphoreType.DMA,
            pltpu.SemaphoreType.DMA,
        ],
    )
    return f(s_mat, cm_mat)


def kernel(ref, query):
    s_mat, cm_mat = _scores(jnp.swapaxes(ref, 1, 2), query)
    d_flat, i_flat = _sc_topk(s_mat, cm_mat)
    dist = d_flat.reshape(B, Q, K)
    idx = i_flat.reshape(B, Q, K)
    return dist, idx.astype(jnp.int64)
